# Initial kernel scaffold; baseline (speedup 1.0000x reference)
import jax
import jax.numpy as jnp
from jax.experimental import pallas as pl


def kernel(x, gid, w_mu, b_mu, w0_mu, b0_mu):
    raise NotImplementedError("write your pallas kernel here")


# speedup vs baseline: 2.8680x; 2.8680x over previous
import jax
import jax.numpy as jnp
from jax.experimental import pallas as pl
from jax.experimental.pallas import tpu as pltpu

B, F, U, G = 1024, 256, 256, 64


def _dense_kernel(onehot_ref, x_ref, w_ref, b_ref, w0_ref, b0_ref,
                  out_ref, loss_ref):
    g = pl.program_id(0)
    w = w_ref[0]
    bias = b_ref[0, 0]
    y = jax.lax.dot_general(
        x_ref[...], w, (((1,), (1,)), ((), ())),
        preferred_element_type=jnp.float32,
        precision=jax.lax.Precision.HIGHEST)
    m = onehot_ref[0] > 0.5
    val = jax.nn.relu(y + bias[None, :])
    cnt = jnp.sum(onehot_ref[0])
    wl = jnp.sum(jnp.square(w - w0_ref[0]))
    bl = jnp.sum(jnp.square(bias - b0_ref[0]))
    step_loss = jnp.full((1, 1), cnt * (wl + bl), dtype=jnp.float32)

    @pl.when(g == 0)
    def _():
        out_ref[...] = jnp.where(m, val, 0.0)
        loss_ref[...] = step_loss

    @pl.when(g != 0)
    def _():
        out_ref[...] = jnp.where(m, val, out_ref[...])
        loss_ref[...] = loss_ref[...] + step_loss


def kernel(x, gid, w_mu, b_mu, w0_mu, b0_mu):
    onehot = (jnp.arange(G, dtype=jnp.int32)[:, None] == gid[None, :])
    onehot = onehot.astype(jnp.float32)[:, :, None]

    out, loss = pl.pallas_call(
        _dense_kernel,
        grid=(G,),
        in_specs=[
            pl.BlockSpec((1, B, 1), lambda g: (g, 0, 0)),
            pl.BlockSpec((B, F), lambda g: (0, 0)),
            pl.BlockSpec((1, U, F), lambda g: (g, 0, 0)),
            pl.BlockSpec((1, 1, U), lambda g: (g, 0, 0)),
            pl.BlockSpec((1, U, F), lambda g: (0, 0, 0)),
            pl.BlockSpec((1, U), lambda g: (0, 0)),
        ],
        out_specs=[
            pl.BlockSpec((B, U), lambda g: (0, 0)),
            pl.BlockSpec((1, 1), lambda g: (0, 0)),
        ],
        out_shape=[
            jax.ShapeDtypeStruct((B, U), jnp.float32),
            jax.ShapeDtypeStruct((1, 1), jnp.float32),
        ],
        compiler_params=pltpu.CompilerParams(
            dimension_semantics=("arbitrary",)),
    )(onehot, x, w_mu, b_mu[:, None, :], w0_mu, b0_mu)
    return out, loss[0, 0]


# speedup vs baseline: 3.8959x; 1.3584x over previous
import jax
import jax.numpy as jnp
from jax.experimental import pallas as pl
from jax.experimental.pallas import tpu as pltpu

B, F, U, G = 1024, 256, 256, 64


def _dense_kernel(onehot_ref, x_ref, w_ref, b_ref, w0_ref, b0_ref,
                  out_ref, loss_ref):
    g = pl.program_id(0)
    w = w_ref[0]
    bias = b_ref[0, 0]
    y = jax.lax.dot_general(
        x_ref[...].astype(jnp.bfloat16), w.astype(jnp.bfloat16),
        (((1,), (1,)), ((), ())),
        preferred_element_type=jnp.float32)
    m = onehot_ref[0] > 0.5
    val = jax.nn.relu(y + bias[None, :])
    cnt = jnp.sum(onehot_ref[0])
    wl = jnp.sum(jnp.square(w - w0_ref[0]))
    bl = jnp.sum(jnp.square(bias - b0_ref[0]))
    step_loss = jnp.full((1, 1), cnt * (wl + bl), dtype=jnp.float32)

    @pl.when(g == 0)
    def _():
        out_ref[...] = jnp.where(m, val, 0.0)
        loss_ref[...] = step_loss

    @pl.when(g != 0)
    def _():
        out_ref[...] = jnp.where(m, val, out_ref[...])
        loss_ref[...] = loss_ref[...] + step_loss


def kernel(x, gid, w_mu, b_mu, w0_mu, b0_mu):
    onehot = (jnp.arange(G, dtype=jnp.int32)[:, None] == gid[None, :])
    onehot = onehot.astype(jnp.float32)[:, :, None]

    out, loss = pl.pallas_call(
        _dense_kernel,
        grid=(G,),
        in_specs=[
            pl.BlockSpec((1, B, 1), lambda g: (g, 0, 0)),
            pl.BlockSpec((B, F), lambda g: (0, 0)),
            pl.BlockSpec((1, U, F), lambda g: (g, 0, 0)),
            pl.BlockSpec((1, 1, U), lambda g: (g, 0, 0)),
            pl.BlockSpec((1, U, F), lambda g: (0, 0, 0)),
            pl.BlockSpec((1, U), lambda g: (0, 0)),
        ],
        out_specs=[
            pl.BlockSpec((B, U), lambda g: (0, 0)),
            pl.BlockSpec((1, 1), lambda g: (0, 0)),
        ],
        out_shape=[
            jax.ShapeDtypeStruct((B, U), jnp.float32),
            jax.ShapeDtypeStruct((1, 1), jnp.float32),
        ],
        compiler_params=pltpu.CompilerParams(
            dimension_semantics=("arbitrary",)),
    )(onehot, x, w_mu, b_mu[:, None, :], w0_mu, b0_mu)
    return out, loss[0, 0]


# speedup vs baseline: 5.7602x; 1.4785x over previous
import functools

import jax
import jax.numpy as jnp
from jax.experimental import pallas as pl
from jax.experimental.pallas import tpu as pltpu
from jax.experimental.pallas import tpu_sc as plsc

B, F, U, G = 1024, 256, 256, 64
NP = 3072
CH = 32
HG = G // 2
SC_WIN = 128


def _route_kernel(gid_ref, posp_ref, counts_ref, padoff_ref):
    gid = gid_ref[...]
    iota_g = jax.lax.broadcasted_iota(jnp.int32, (G, B), 0)
    onehot = (gid == iota_g).astype(jnp.int32)

    csum = onehot
    k = 1
    while k < B:
        shifted = jnp.concatenate(
            [jnp.zeros((G, k), jnp.int32), csum[:, :B - k]], axis=1)
        csum = csum + shifted
        k *= 2

    counts = csum[:, B - 1:B]
    pc = ((jnp.maximum(counts, 1) + 31) >> 5) << 5
    ex = jnp.concatenate([jnp.zeros((1, 1), jnp.int32), pc[:G - 1]], axis=0)
    k = 1
    while k < G:
        shifted = jnp.concatenate(
            [jnp.zeros((k, 1), jnp.int32), ex[:G - k]], axis=0)
        ex = ex + shifted
        k *= 2
    padoff = ex

    posp = jnp.sum(onehot * (padoff + csum - 1), axis=0, keepdims=True)
    posp_ref[...] = posp
    counts_ref[...] = counts
    padoff_ref[...] = padoff


def _route(gid_row):
    return pl.pallas_call(
        _route_kernel,
        in_specs=[pl.BlockSpec((1, B), lambda: (0, 0))],
        out_specs=[
            pl.BlockSpec((1, B), lambda: (0, 0)),
            pl.BlockSpec((G, 1), lambda: (0, 0)),
            pl.BlockSpec((G, 1), lambda: (0, 0)),
        ],
        out_shape=[
            jax.ShapeDtypeStruct((1, B), jnp.int32),
            jax.ShapeDtypeStruct((G, 1), jnp.int32),
            jax.ShapeDtypeStruct((G, 1), jnp.int32),
        ],
    )(gid_row)


def _sc_mesh():
    return plsc.VectorSubcoreMesh(core_axis_name="c", subcore_axis_name="s")


def _sc_scatter_rows(x, idx_row):
    @pl.kernel(out_type=jax.ShapeDtypeStruct((NP, F), jnp.float32),
               mesh=_sc_mesh())
    def k(x_hbm, i_hbm, o_hbm):
        def body(x_vmem, i_vmem):
            pltpu.sync_copy(x_vmem, o_hbm.at[i_vmem.at[0]])

        pltpu.emit_pipeline(
            body,
            grid=(B // SC_WIN,),
            in_specs=[
                pl.BlockSpec((SC_WIN, F), lambda i: (i, 0)),
                pl.BlockSpec((1, SC_WIN), lambda i: (0, i)),
            ],
            out_specs=[],
            core_axis_name="s",
            dimension_semantics=(pltpu.PARALLEL,),
        )(x_hbm, i_hbm)

    return k(x, idx_row)


def _sc_gather_rows(yp, idx_row):
    @pl.kernel(out_type=jax.ShapeDtypeStruct((B, U), jnp.float32),
               mesh=_sc_mesh())
    def k(y_hbm, i_hbm, o_hbm):
        def body(i_vmem, o_vmem):
            pltpu.sync_copy(y_hbm.at[i_vmem.at[0]], o_vmem)

        pltpu.emit_pipeline(
            body,
            grid=(B // SC_WIN,),
            in_specs=[pl.BlockSpec((1, SC_WIN), lambda i: (0, i))],
            out_specs=[pl.BlockSpec((SC_WIN, U), lambda i: (i, 0))],
            core_axis_name="s",
            dimension_semantics=(pltpu.PARALLEL,),
        )(i_hbm, o_hbm)

    return k(yp, idx_row)


def _mm_kernel(padoff_ref, counts_ref, xp_ref, w_ref, b_ref, w0_ref, b0_ref,
               yp_ref, loss_ref):
    h = pl.program_id(0)
    w0 = w0_ref[0]
    b0 = b0_ref[...]

    def do_chunk(j, off):
        off = pl.multiple_of(off, CH)
        xs = xp_ref[pl.ds(off, CH), :].astype(jnp.bfloat16)
        wj = w_ref[j].astype(jnp.bfloat16)
        y = jax.lax.dot_general(xs, wj, (((1,), (1,)), ((), ())),
                                preferred_element_type=jnp.float32)
        yp_ref[pl.ds(off, CH), :] = jnp.maximum(y + b_ref[j][None, :], 0.0)

    accw = jnp.zeros((1, U), jnp.float32)
    accb = jnp.zeros((1, U), jnp.float32)
    for j in range(HG):
        gidx = h * HG + j
        off = padoff_ref[gidx]
        cnt = counts_ref[gidx]
        cntf = cnt.astype(jnp.float32)
        do_chunk(j, off)
        nch = (jnp.maximum(cnt, 1) + 31) >> 5

        def ov_body(c, carry):
            do_chunk(j, off + c * CH)
            return carry

        jax.lax.fori_loop(1, nch, ov_body, 0)
        dw = w_ref[j] - w0
        accw = accw + cntf * jnp.sum(dw * dw, axis=0, keepdims=True)
        db = b_ref[j][None, :] - b0
        accb = accb + cntf * (db * db)

    step_loss = jnp.full((1, 1), jnp.sum(accw) + jnp.sum(accb), jnp.float32)

    @pl.when(h == 0)
    def _():
        loss_ref[...] = step_loss

    @pl.when(h != 0)
    def _():
        loss_ref[...] = loss_ref[...] + step_loss


def _grouped_matmul(padoff, counts, xp, w_mu, b_mu, w0_mu, b0_mu):
    grid_spec = pltpu.PrefetchScalarGridSpec(
        num_scalar_prefetch=2,
        grid=(2,),
        in_specs=[
            pl.BlockSpec((NP, F), lambda h, *_: (0, 0)),
            pl.BlockSpec((HG, U, F), lambda h, *_: (h, 0, 0)),
            pl.BlockSpec((HG, U), lambda h, *_: (h, 0)),
            pl.BlockSpec((1, U, F), lambda h, *_: (0, 0, 0)),
            pl.BlockSpec((1, U), lambda h, *_: (0, 0)),
        ],
        out_specs=[
            pl.BlockSpec((NP, U), lambda h, *_: (0, 0)),
            pl.BlockSpec((1, 1), lambda h, *_: (0, 0)),
        ],
    )
    return pl.pallas_call(
        _mm_kernel,
        grid_spec=grid_spec,
        out_shape=[
            jax.ShapeDtypeStruct((NP, U), jnp.float32),
            jax.ShapeDtypeStruct((1, 1), jnp.float32),
        ],
        compiler_params=pltpu.CompilerParams(
            dimension_semantics=("arbitrary",)),
    )(padoff, counts, xp, w_mu, b_mu, w0_mu, b0_mu)


def kernel(x, gid, w_mu, b_mu, w0_mu, b0_mu):
    posp, counts, padoff = _route(gid[None, :])
    xp = _sc_scatter_rows(x, posp)
    yp, loss = _grouped_matmul(padoff[:, 0], counts[:, 0], xp,
                               w_mu, b_mu, w0_mu, b0_mu)
    out = _sc_gather_rows(yp, posp)
    return out, loss[0, 0]


# speedup vs baseline: 6.4165x; 1.1139x over previous
import jax
import jax.numpy as jnp
from jax.experimental import pallas as pl
from jax.experimental.pallas import tpu as pltpu
from jax.experimental.pallas import tpu_sc as plsc

B, F, U, G = 1024, 256, 256, 64
CH = 32
L1 = G * CH
NP = L1 + 1024
MM_STEPS = 4
HG = G // MM_STEPS
LG = 8
SC_WIN = 128


def _route_kernel(gid_ref, posp_ref, counts_ref, ovoff_ref):
    gid = gid_ref[...]
    iota_g = jax.lax.broadcasted_iota(jnp.int32, (G, B), 0)
    onehot = (gid == iota_g).astype(jnp.int32)

    csum = onehot
    k = 1
    while k < B:
        shifted = jnp.concatenate(
            [jnp.zeros((G, k), jnp.int32), csum[:, :B - k]], axis=1)
        csum = csum + shifted
        k *= 2

    counts = csum[:, B - 1:B]
    ovc = jnp.maximum(counts - CH, 0)
    ovpc = ((ovc + 31) >> 5) << 5
    ex = jnp.concatenate([jnp.zeros((1, 1), jnp.int32), ovpc[:G - 1]], axis=0)
    k = 1
    while k < G:
        shifted = jnp.concatenate(
            [jnp.zeros((k, 1), jnp.int32), ex[:G - k]], axis=0)
        ex = ex + shifted
        k *= 2
    ovoff = ex + L1

    rank = csum - 1
    slot = jnp.where(rank < CH,
                     CH * iota_g + rank,
                     ovoff + rank - CH)
    posp = jnp.sum(onehot * slot, axis=0, keepdims=True)
    posp_ref[...] = posp
    counts_ref[...] = counts
    ovoff_ref[...] = ovoff


def _route(gid_row):
    return pl.pallas_call(
        _route_kernel,
        in_specs=[pl.BlockSpec((1, B), lambda: (0, 0))],
        out_specs=[
            pl.BlockSpec((1, B), lambda: (0, 0)),
            pl.BlockSpec((G, 1), lambda: (0, 0)),
            pl.BlockSpec((G, 1), lambda: (0, 0)),
        ],
        out_shape=[
            jax.ShapeDtypeStruct((1, B), jnp.int32),
            jax.ShapeDtypeStruct((G, 1), jnp.int32),
            jax.ShapeDtypeStruct((G, 1), jnp.int32),
        ],
    )(gid_row)


def _sc_mesh():
    return plsc.VectorSubcoreMesh(core_axis_name="c", subcore_axis_name="s")


def _sc_scatter_rows(x, idx_row):
    @pl.kernel(out_type=jax.ShapeDtypeStruct((NP, F), jnp.float32),
               mesh=_sc_mesh())
    def k(x_hbm, i_hbm, o_hbm):
        def body(x_vmem, i_vmem):
            pltpu.sync_copy(x_vmem, o_hbm.at[i_vmem.at[0]])

        pltpu.emit_pipeline(
            body,
            grid=(B // SC_WIN,),
            in_specs=[
                pl.BlockSpec((SC_WIN, F), lambda i: (i, 0)),
                pl.BlockSpec((1, SC_WIN), lambda i: (0, i)),
            ],
            out_specs=[],
            core_axis_name=("c", "s"),
            dimension_semantics=(pltpu.PARALLEL,),
        )(x_hbm, i_hbm)

    return k(x, idx_row)


def _sc_gather_rows(yp, idx_row):
    @pl.kernel(out_type=jax.ShapeDtypeStruct((B, U), jnp.float32),
               mesh=_sc_mesh())
    def k(y_hbm, i_hbm, o_hbm):
        def body(i_vmem, o_vmem):
            pltpu.sync_copy(y_hbm.at[i_vmem.at[0]], o_vmem)

        pltpu.emit_pipeline(
            body,
            grid=(B // SC_WIN,),
            in_specs=[pl.BlockSpec((1, SC_WIN), lambda i: (0, i))],
            out_specs=[pl.BlockSpec((SC_WIN, U), lambda i: (i, 0))],
            core_axis_name=("c", "s"),
            dimension_semantics=(pltpu.PARALLEL,),
        )(i_hbm, o_hbm)

    return k(yp, idx_row)


def _mm_kernel(counts_ref, ovoff_ref, xp1_ref, xpov_ref, w_ref, b_ref,
               yp_ref):
    h = pl.program_id(0)
    ybase = pl.multiple_of(h * (HG * CH), HG * CH)

    for j in range(HG):
        xs = xp1_ref[CH * j:CH * (j + 1), :].astype(jnp.bfloat16)
        y = jax.lax.dot_general(
            xs, w_ref[j].astype(jnp.bfloat16),
            (((1,), (1,)), ((), ())),
            preferred_element_type=jnp.float32)
        yv = jnp.maximum(y + b_ref[j][None, :], 0.0)
        yp_ref[pl.ds(ybase + CH * j, CH), :] = yv

    for j in range(HG):
        cnt = counts_ref[h * HG + j]
        novf = (jnp.maximum(cnt - CH, 0) + CH - 1) // CH
        ov0 = ovoff_ref[h * HG + j]

        def ov_body(c, carry, j=j, ov0=ov0):
            src = pl.multiple_of(ov0 - L1 + CH * c, CH)
            dst = pl.multiple_of(ov0 + CH * c, CH)
            xs = xpov_ref[pl.ds(src, CH), :].astype(jnp.bfloat16)
            y = jax.lax.dot_general(xs, w_ref[j].astype(jnp.bfloat16),
                                    (((1,), (1,)), ((), ())),
                                    preferred_element_type=jnp.float32)
            yp_ref[pl.ds(dst, CH), :] = jnp.maximum(
                y + b_ref[j][None, :], 0.0)
            return carry

        jax.lax.fori_loop(0, novf, ov_body, 0)


def _grouped_matmul(counts, ovoff, xp, w_mu, b_mu):
    grid_spec = pltpu.PrefetchScalarGridSpec(
        num_scalar_prefetch=2,
        grid=(MM_STEPS,),
        in_specs=[
            pl.BlockSpec((HG * CH, F), lambda h, *_: (h, 0)),
            pl.BlockSpec((NP - L1, F), lambda h, *_: (2, 0)),
            pl.BlockSpec((HG, U, F), lambda h, *_: (h, 0, 0)),
            pl.BlockSpec((HG, U), lambda h, *_: (h, 0)),
        ],
        out_specs=[
            pl.BlockSpec((NP, U), lambda h, *_: (0, 0)),
        ],
    )
    return pl.pallas_call(
        _mm_kernel,
        grid_spec=grid_spec,
        out_shape=[
            jax.ShapeDtypeStruct((NP, U), jnp.float32),
        ],
        compiler_params=pltpu.CompilerParams(
            dimension_semantics=("arbitrary",)),
    )(counts, ovoff, xp, xp, w_mu, b_mu)[0]


def _loss_kernel(counts_ref, w_ref, b_ref, w0_ref, b0_ref, loss_ref):
    h = pl.program_id(0)
    w0 = w0_ref[0]
    b0 = b0_ref[...]
    accw = jnp.zeros((1, U), jnp.float32)
    accb = jnp.zeros((1, U), jnp.float32)
    for j in range(LG):
        cntf = counts_ref[h * LG + j].astype(jnp.float32)
        dw = w_ref[j] - w0
        accw = accw + cntf * jnp.sum(dw * dw, axis=0, keepdims=True)
        db = b_ref[j][None, :] - b0
        accb = accb + cntf * (db * db)
    step_loss = jnp.full((1, 1), jnp.sum(accw) + jnp.sum(accb), jnp.float32)

    @pl.when(h == 0)
    def _():
        loss_ref[...] = step_loss

    @pl.when(h != 0)
    def _():
        loss_ref[...] = loss_ref[...] + step_loss


def _prior_loss(counts, w_mu, b_mu, w0_mu, b0_mu):
    grid_spec = pltpu.PrefetchScalarGridSpec(
        num_scalar_prefetch=1,
        grid=(G // LG,),
        in_specs=[
            pl.BlockSpec((LG, U, F), lambda h, *_: (h, 0, 0)),
            pl.BlockSpec((LG, U), lambda h, *_: (h, 0)),
            pl.BlockSpec((1, U, F), lambda h, *_: (0, 0, 0)),
            pl.BlockSpec((1, U), lambda h, *_: (0, 0)),
        ],
        out_specs=[pl.BlockSpec((1, 1), lambda h, *_: (0, 0))],
    )
    return pl.pallas_call(
        _loss_kernel,
        grid_spec=grid_spec,
        out_shape=[jax.ShapeDtypeStruct((1, 1), jnp.float32)],
        compiler_params=pltpu.CompilerParams(
            dimension_semantics=("arbitrary",)),
    )(counts, w_mu, b_mu, w0_mu, b0_mu)[0]


def kernel(x, gid, w_mu, b_mu, w0_mu, b0_mu):
    posp, counts, ovoff = _route(gid[None, :])
    xp = _sc_scatter_rows(x, posp)
    yp = _grouped_matmul(counts[:, 0], ovoff[:, 0], xp, w_mu, b_mu)
    loss = _prior_loss(counts[:, 0], w_mu, b_mu, w0_mu, b0_mu)
    out = _sc_gather_rows(yp, posp)
    return out, loss[0, 0]
